# baseline (speedup 1.0000x reference)
import functools

import jax
import jax.numpy as jnp
from jax.experimental import pallas as pl
from jax.experimental.pallas import tpu as pltpu

N_TOK = 4096
D = 1024
E = 6
NC = 1000
BT = 512
NEG = -1e30


def _top2(gl):
    lane = jax.lax.broadcasted_iota(jnp.int32, gl.shape, 1)
    m1 = jnp.max(gl, axis=-1, keepdims=True)
    i1 = jnp.min(jnp.where(gl == m1, lane, 127), axis=-1, keepdims=True)
    gl2 = jnp.where(lane == i1, NEG, gl)
    m2 = jnp.max(gl2, axis=-1, keepdims=True)
    i2 = jnp.min(jnp.where(gl2 == m2, lane, 127), axis=-1, keepdims=True)
    s = jnp.sum(jnp.exp(gl - m1), axis=-1, keepdims=True)
    v1 = 1.0 / s
    v2 = jnp.exp(m2 - m1) / s
    denom = v1 + v2 + 1e-9
    return i1, i2, v1 / denom, v2 / denom


def _fused_body(x_ref, w1_ref, w2_ref, wg_ref, we_ref, be_ref, b1_ref,
                b2_ref, wp_ref, bp_ref, out_ref):
    f32 = jnp.float32
    xb = x_ref[...]
    h1 = jnp.dot(xb, w1_ref[...], preferred_element_type=f32) + b1_ref[...]
    h1 = jnp.maximum(h1, 0.0).astype(jnp.bfloat16)
    h2 = jnp.dot(h1, w2_ref[...], preferred_element_type=f32) + b2_ref[...]
    h2 = jnp.maximum(h2, 0.0)
    h2b = h2.astype(jnp.bfloat16)
    gl = jnp.dot(h2, wg_ref[...], preferred_element_type=f32)
    lane = jax.lax.broadcasted_iota(jnp.int32, gl.shape, 1)
    gl = jnp.where(lane < E, gl, NEG)
    i1, i2, w1w, w2w = _top2(gl)
    acc = jnp.zeros((xb.shape[0], D), f32)
    for e in range(E):
        we = (jnp.where(i1 == e, w1w, 0.0) + jnp.where(i2 == e, w2w, 0.0))
        eo = jnp.dot(h2b, we_ref[e], preferred_element_type=f32) + be_ref[e][None, :]
        acc = acc + we * eo
    logits = jnp.dot(acc.astype(jnp.bfloat16), wp_ref[...],
                     preferred_element_type=f32) + bp_ref[...]
    m = jnp.max(logits, axis=-1, keepdims=True)
    lse = jnp.log(jnp.sum(jnp.exp(logits - m), axis=-1, keepdims=True))
    out_ref[...] = logits - m - lse


@jax.jit
def kernel(x, W1, b1, W2, b2, Wg, We, be, Wp, bp):
    bf16 = jnp.bfloat16
    xf = x.reshape(N_TOK, D).astype(bf16)
    wg_pad = jnp.zeros((D, 128), jnp.float32).at[:, :E].set(Wg)
    wp_pad = Wp.astype(bf16)
    bp_pad = bp.reshape(1, NC)

    full = lambda s: pl.BlockSpec(s, lambda i: tuple(0 for _ in s))
    out = pl.pallas_call(
        _fused_body,
        grid=(N_TOK // BT,),
        in_specs=[
            pl.BlockSpec((BT, D), lambda i: (i, 0)),
            full((D, D)), full((D, D)), full((D, 128)),
            full((E, D, D)), full((E, D)),
            full((1, D)), full((1, D)),
            full((D, NC)), full((1, NC)),
        ],
        out_specs=pl.BlockSpec((BT, NC), lambda i: (i, 0)),
        out_shape=jax.ShapeDtypeStruct((N_TOK, NC), jnp.float32),
        compiler_params=pltpu.CompilerParams(
            dimension_semantics=("arbitrary",)),
    )(xf, W1.astype(bf16), W2.astype(bf16), wg_pad, We.astype(bf16),
      be, b1.reshape(1, D), b2.reshape(1, D), wp_pad, bp_pad)
    return out
